# baseline (speedup 1.0000x reference)
import functools

import numpy as np

import jax
import jax.numpy as jnp
from jax import lax
from jax.experimental import pallas as pl
from jax.experimental.pallas import tpu as pltpu
from jax.experimental.pallas import tpu_sc as plsc

N = 10000
E = 160000
F = 128
HEADS = 10
NPAD = 10240
NC = 2
NS = 16
NTILES = NC * NS
BB = 128
NB = 42
EPAD = NTILES * NB * BB
FH = 64
BM = 256
RPT = NPAD // NS
NZ = RPT // BB

_PERM = np.concatenate(
    [np.stack([np.arange(16) + 32 * k2, np.arange(16) + 32 * k2 + 16],
              axis=1).reshape(-1) for k2 in range(2)])


def _make_edge_sc(H):
    mesh = plsc.VectorSubcoreMesh(core_axis_name="c", subcore_axis_name="s")

    @functools.partial(
        pl.kernel,
        out_type=(
            jax.ShapeDtypeStruct((NC, 2 * H, NPAD, 80), jnp.float32),
            jax.ShapeDtypeStruct((NC, NPAD, 16), jnp.float32),
            jax.ShapeDtypeStruct((NTILES, NB, BB, 16), jnp.float32),
        ),
        mesh=mesh,
        compiler_params=pltpu.CompilerParams(use_tc_tiling_on_sc=False),
        scratch_types=[
            pltpu.VMEM((NB, BB), jnp.int32),
            pltpu.VMEM((NB, BB), jnp.int32),
            pltpu.VMEM((BB, 16), jnp.float32),
            pltpu.VMEM((BB, 16), jnp.float32),
            pltpu.VMEM((BB, 16), jnp.float32),
            pltpu.VMEM((BB, 16), jnp.float32),
            pltpu.VMEM((BB, FH), jnp.float32),
            pltpu.VMEM((BB, FH), jnp.float32),
            pltpu.VMEM((BB, 80), jnp.float32),
            pltpu.VMEM((BB, 80), jnp.float32),
            pltpu.VMEM((BB, 80), jnp.float32),
            pltpu.VMEM((BB, 16), jnp.float32),
            pltpu.VMEM_SHARED((NPAD, 80), jnp.float32),
            pltpu.VMEM_SHARED((NPAD, 16), jnp.float32),
            pltpu.SemaphoreType.DMA,
            pltpu.SemaphoreType.DMA,
            pltpu.SemaphoreType.DMA,
            pltpu.SemaphoreType.DMA,
            pltpu.SemaphoreType.DMA,
            pltpu.SemaphoreType.DMA,
            pltpu.SemaphoreType.DMA,
            pltpu.SemaphoreType.DMA,
        ],
    )
    def edge_kernel(feat, a_s, a_d, srcs, dsts, partsF, partsD, wout,
                    idx_s, idx_d, ar, ad, wb0, wb1, rbf0, rbf1, rfp0, rfp1,
                    zbuf, zbufd, accF, accD,
                    semr0, semr1, semw0, semw1, sems0, sems1, semd0, semd1):
        cid = lax.axis_index("c")
        sid = lax.axis_index("s")
        tid = cid * NS + sid
        fzero = jnp.zeros((16,), jnp.float32)

        def zb(j, _):
            for k2 in range(80 // 16):
                zbuf[j, k2 * 16:(k2 + 1) * 16] = fzero
            rfp0[j, 64:80] = fzero
            rfp1[j, 64:80] = fzero
            zbufd[j, :] = fzero
            return 0
        lax.fori_loop(0, BB, zb, 0)

        pltpu.sync_copy(srcs.at[tid], idx_s)
        pltpu.sync_copy(dsts.at[tid], idx_d)

        def batch_a(b, _):
            ga = pltpu.async_copy(a_s.at[idx_s.at[b]], ar, semr0)
            gb = pltpu.async_copy(a_d.at[idx_d.at[b]], ad, semr1)
            ga.wait()
            gb.wait()

            def row_a(j, _):
                e = ar[j, :] + ad[j, :]
                e = jnp.where(e >= 0.0, e, e * 0.2)
                wb0[j, :] = jnp.exp(e)
                return 0
            lax.fori_loop(0, BB, row_a, 0, unroll=4)
            pltpu.sync_copy(wb0, wout.at[tid, b])
            return 0
        lax.fori_loop(0, NB, batch_a, 0)

        def head(hh, _):
            even = hh % 2 == 0

            def zc(i, _):
                pltpu.sync_copy(zbuf, accF.at[pl.ds(sid * RPT + i * BB, BB)])
                return 0
            lax.fori_loop(0, NZ, zc, 0)

            @pl.when(hh == 0)
            def _():
                def zcd(i, _):
                    pltpu.sync_copy(
                        zbufd, accD.at[pl.ds(sid * RPT + i * BB, BB)])
                    return 0
                lax.fori_loop(0, NZ, zcd, 0)
            plsc.subcore_barrier()

            hvec = jnp.full((16,), hh // 2, jnp.int32)

            def start(b, rbuf, wbuf, sr, sw):
                pltpu.async_copy(wout.at[tid, b], wbuf, sw)
                pltpu.async_copy(feat.at[hh].at[idx_s.at[b]], rbuf, sr)

            def scale(b, rbuf, wbuf, fbuf, sr, sw):
                pltpu.make_async_copy(wout.at[tid, b], wbuf, sw).wait()
                pltpu.make_async_copy(
                    feat.at[hh].at[idx_s.at[b]], rbuf, sr).wait()

                def row_b(j, _):
                    wrow = wbuf[j, :]
                    ws = wrow.at[hvec].get(mode="promise_in_bounds")
                    for k2 in range(FH // 16):
                        sl = slice(16 * k2, 16 * k2 + 16)
                        fbuf[j, sl] = rbuf[j, sl] * ws
                    return 0
                lax.fori_loop(0, BB, row_b, 0, unroll=4)

            def scat(b, fbuf, wbuf):
                pltpu.sync_copy(fbuf, accF.at[idx_d.at[b]], add=True)

                @pl.when(hh == 0)
                def _():
                    pltpu.sync_copy(wbuf, accD.at[idx_d.at[b]], add=True)

            start(0, rbf0, wb0, semr0, semw0)

            def pair(bb, _):
                b0 = 2 * bb
                start(b0 + 1, rbf1, wb1, semr1, semw1)
                scale(b0, rbf0, wb0, rfp0, semr0, semw0)
                scat(b0, rfp0, wb0)
                scale(b0 + 1, rbf1, wb1, rfp1, semr1, semw1)

                @pl.when(b0 + 2 < NB)
                def _():
                    start(b0 + 2, rbf0, wb0, semr0, semw0)
                scat(b0 + 1, rfp1, wb1)
                return 0
            lax.fori_loop(0, NB // 2, pair, 0)
            plsc.subcore_barrier()

            def dc(i, _):
                base = sid * RPT + i * BB
                pltpu.sync_copy(accF.at[pl.ds(base, BB)], rfp0)
                pltpu.sync_copy(rfp0, partsF.at[cid, hh].at[pl.ds(base, BB)])
                return 0
            lax.fori_loop(0, NZ, dc, 0)

            @pl.when(hh == 0)
            def _():
                def dcd(i, _):
                    base = sid * RPT + i * BB
                    pltpu.sync_copy(accD.at[pl.ds(base, BB)], wb0)
                    pltpu.sync_copy(wb0, partsD.at[cid].at[pl.ds(base, BB)])
                    return 0
                lax.fori_loop(0, NZ, dcd, 0)
            plsc.subcore_barrier()
            return 0
        lax.fori_loop(0, 2 * H, head, 0)

    return edge_kernel


def _feat1_k(x_ref, w_ref, o_ref):
    mm = jnp.dot(x_ref[...], w_ref[...], preferred_element_type=jnp.float32)
    for hh in range(2 * HEADS):
        o_ref[hh] = mm[:, hh * FH:(hh + 1) * FH]


def _feat1(xp, w1c):
    return pl.pallas_call(
        _feat1_k,
        grid=(NPAD // BM,),
        in_specs=[
            pl.BlockSpec((BM, F), lambda i: (i, 0)),
            pl.BlockSpec((F, 2 * HEADS * FH), lambda i: (0, 0)),
        ],
        out_specs=pl.BlockSpec((2 * HEADS, BM, FH), lambda i: (0, i, 0)),
        out_shape=jax.ShapeDtypeStruct((2 * HEADS, NPAD, FH), jnp.float32),
    )(xp, w1c)


def _alpha_k(x_ref, ws_ref, wd_ref, os_ref, od_ref):
    os_ref[...] = jnp.dot(x_ref[...], ws_ref[...],
                          preferred_element_type=jnp.float32)
    od_ref[...] = jnp.dot(x_ref[...], wd_ref[...],
                          preferred_element_type=jnp.float32)


def _alphas(xp, wsp, wdp):
    return pl.pallas_call(
        _alpha_k,
        grid=(NPAD // BM,),
        in_specs=[
            pl.BlockSpec((BM, F), lambda i: (i, 0)),
            pl.BlockSpec((F, 16), lambda i: (0, 0)),
            pl.BlockSpec((F, 16), lambda i: (0, 0)),
        ],
        out_specs=[
            pl.BlockSpec((BM, 16), lambda i: (i, 0)),
            pl.BlockSpec((BM, 16), lambda i: (i, 0)),
        ],
        out_shape=[
            jax.ShapeDtypeStruct((NPAD, 16), jnp.float32),
            jax.ShapeDtypeStruct((NPAD, 16), jnp.float32),
        ],
    )(xp, wsp, wdp)


def _comb1_k(pf_ref, pd_ref, b1_ref, w2_ref, a2s_ref, a2d_ref, p64_ref,
             of_ref, os_ref, od_ref):
    acc = jnp.zeros((BM, F), jnp.float32)
    for h in range(HEADS):
        e0 = pf_ref[0, 2 * h, :, :FH] + pf_ref[1, 2 * h, :, :FH]
        e1 = pf_ref[0, 2 * h + 1, :, :FH] + pf_ref[1, 2 * h + 1, :, :FH]
        den = pd_ref[0, :, h:h + 1] + pd_ref[1, :, h:h + 1] + 1e-16
        v = jnp.concatenate([e0, e1], axis=1) / den
        v = v + b1_ref[0, h]
        v = jnp.where(v > 0, v, jnp.exp(v) - 1.0)
        acc = acc + jnp.dot(v, w2_ref[h], preferred_element_type=jnp.float32)
    p64 = p64_ref[...]
    f_even = jnp.dot(acc[:, :FH], p64, preferred_element_type=jnp.float32)
    f_odd = jnp.dot(acc[:, FH:], p64, preferred_element_type=jnp.float32)
    of_ref[...] = jnp.stack([f_even, f_odd], axis=0)
    s = jnp.sum(acc * a2s_ref[...], axis=1, keepdims=True)
    d = jnp.sum(acc * a2d_ref[...], axis=1, keepdims=True)
    za = jnp.zeros((BM, 15), jnp.float32)
    os_ref[...] = jnp.concatenate([s, za], axis=1)
    od_ref[...] = jnp.concatenate([d, za], axis=1)


def _comb1(partsF, partsD, b1r, w2r, a2s, a2d, p64):
    return pl.pallas_call(
        _comb1_k,
        grid=(NPAD // BM,),
        in_specs=[
            pl.BlockSpec((NC, 2 * HEADS, BM, 80), lambda i: (0, 0, i, 0)),
            pl.BlockSpec((NC, BM, 16), lambda i: (0, i, 0)),
            pl.BlockSpec((1, HEADS, F), lambda i: (0, 0, 0)),
            pl.BlockSpec((HEADS, F, F), lambda i: (0, 0, 0)),
            pl.BlockSpec((1, F), lambda i: (0, 0)),
            pl.BlockSpec((1, F), lambda i: (0, 0)),
            pl.BlockSpec((FH, FH), lambda i: (0, 0)),
        ],
        out_specs=[
            pl.BlockSpec((2, BM, FH), lambda i: (0, i, 0)),
            pl.BlockSpec((BM, 16), lambda i: (i, 0)),
            pl.BlockSpec((BM, 16), lambda i: (i, 0)),
        ],
        out_shape=[
            jax.ShapeDtypeStruct((2, NPAD, FH), jnp.float32),
            jax.ShapeDtypeStruct((NPAD, 16), jnp.float32),
            jax.ShapeDtypeStruct((NPAD, 16), jnp.float32),
        ],
    )(partsF, partsD, b1r, w2r, a2s, a2d, p64)


def _pool_k(pf_ref, pd_ref, b2_ref, o_ref):
    i = pl.program_id(0)
    e0 = pf_ref[0, 0, :, :FH] + pf_ref[1, 0, :, :FH]
    e1 = pf_ref[0, 1, :, :FH] + pf_ref[1, 1, :, :FH]
    den = pd_ref[0, :, 0:1] + pd_ref[1, :, 0:1] + 1e-16
    v = jnp.concatenate([e0, e1], axis=1) / den
    v = jnp.maximum(v + b2_ref[...], 0.0)
    rid = i * BM + lax.broadcasted_iota(jnp.int32, (BM, 1), 0)
    v = jnp.where(rid < N, v, 0.0)
    m = jnp.max(v.reshape(BM // 8, 8, F), axis=0)

    @pl.when(i == 0)
    def _():
        o_ref[...] = m

    @pl.when(i > 0)
    def _():
        o_ref[...] = jnp.maximum(o_ref[...], m)


def _pool(partsF2, partsD2, b2r):
    return pl.pallas_call(
        _pool_k,
        grid=(NPAD // BM,),
        in_specs=[
            pl.BlockSpec((NC, 2, BM, 80), lambda i: (0, 0, i, 0)),
            pl.BlockSpec((NC, BM, 16), lambda i: (0, i, 0)),
            pl.BlockSpec((1, F), lambda i: (0, 0)),
        ],
        out_specs=pl.BlockSpec((8, F), lambda i: (0, 0)),
        out_shape=jax.ShapeDtypeStruct((8, F), jnp.float32),
    )(partsF2, partsD2, b2r)


def _mlp_k(g_ref, w0_ref, b0_ref, w1_ref, b1_ref, w2_ref, b2_ref,
           w3_ref, b3_ref, o_ref):
    g = jnp.max(g_ref[...], axis=0, keepdims=True)
    g = jnp.maximum(jnp.dot(g, w0_ref[...],
                            preferred_element_type=jnp.float32)
                    + b0_ref[...], 0.0)
    g = jnp.maximum(jnp.dot(g, w1_ref[...],
                            preferred_element_type=jnp.float32)
                    + b1_ref[...], 0.0)
    g = jnp.maximum(jnp.dot(g, w2_ref[...],
                            preferred_element_type=jnp.float32)
                    + b2_ref[...], 0.0)
    o_ref[...] = (jnp.dot(g, w3_ref[...], preferred_element_type=jnp.float32)
                  + b3_ref[...])


def _mlp(g8, fcg_w, fcg_b, fc1_w, fc1_b, fc2_w, fc2_b, out_w, out_b):
    return pl.pallas_call(
        _mlp_k,
        out_shape=jax.ShapeDtypeStruct((1, 128), jnp.float32),
    )(g8, fcg_w, fcg_b.reshape(1, -1), fc1_w, fc1_b.reshape(1, -1),
      fc2_w, fc2_b.reshape(1, -1), out_w, out_b.reshape(1, -1))


def kernel(x, edge_index, W1, a1_src, a1_dst, b1, W2, a2_src, a2_dst, b2,
           fcg_w, fcg_b, fc1_w, fc1_b, fc2_w, fc2_b, out_w, out_b):
    loop = jnp.arange(N, dtype=edge_index.dtype)
    ndum = EPAD - E - N
    dum = N + (jnp.arange(ndum, dtype=jnp.int32) % (NPAD - N))
    src = jnp.concatenate([edge_index[0], loop, dum])
    dst = jnp.concatenate([edge_index[1], loop, dum])
    srcs = src.reshape(NTILES, NB, BB)
    dsts = dst.reshape(NTILES, NB, BB)

    xp = jnp.pad(x, ((0, NPAD - N), (0, 0)))

    w1r = W1.reshape(F, HEADS, F).transpose(1, 0, 2)
    w1h = w1r.reshape(HEADS, F, 2, FH).transpose(0, 2, 1, 3)
    w1c = w1h.reshape(2 * HEADS, F, FH)
    w1c = w1c.transpose(1, 0, 2).reshape(F, 2 * HEADS * FH)
    ws1 = jnp.einsum("fhc,hc->fh", W1.reshape(F, HEADS, F), a1_src)
    wd1 = jnp.einsum("fhc,hc->fh", W1.reshape(F, HEADS, F), a1_dst)
    wsp = jnp.pad(ws1, ((0, 0), (0, 6)))
    wdp = jnp.pad(wd1, ((0, 0), (0, 6)))
    w2r = W2.reshape(HEADS, F, F)
    b1r = b1.reshape(1, HEADS, F)
    b2r = b2.reshape(1, F)
    p64 = jnp.eye(FH, dtype=jnp.float32)

    feat1 = _feat1(xp, w1c)
    a_s1, a_d1 = _alphas(xp, wsp, wdp)
    pF1, pD1, _ = _make_edge_sc(HEADS)(feat1, a_s1, a_d1, srcs, dsts)

    feat2, a_s2, a_d2 = _comb1(pF1, pD1, b1r, w2r, a2_src, a2_dst, p64)

    pF2, pD2, _ = _make_edge_sc(1)(feat2, a_s2, a_d2, srcs, dsts)

    g8 = _pool(pF2, pD2, b2r)
    return _mlp(g8, fcg_w, fcg_b, fc1_w, fc1_b, fc2_w, fc2_b, out_w, out_b)


# speedup vs baseline: 1.5812x; 1.5812x over previous
import functools

import jax
import jax.numpy as jnp
from jax import lax
from jax.experimental import pallas as pl
from jax.experimental.pallas import tpu as pltpu
from jax.experimental.pallas import tpu_sc as plsc

N = 10000
E = 160000
F = 128
HEADS = 10
NPAD = 10240
NC = 2
NS = 16
NTILES = NC * NS
BB = 128
NB = 42
EPAD = NTILES * NB * BB
FH = 64
FW = 80
BM = 256
RPT = NPAD // NS
NZ = RPT // BB


def _make_edge_sc(H):
    mesh = plsc.VectorSubcoreMesh(core_axis_name="c", subcore_axis_name="s")

    @functools.partial(
        pl.kernel,
        out_type=(
            jax.ShapeDtypeStruct((NC, 2 * H, NPAD, FW), jnp.float32),
            jax.ShapeDtypeStruct((NTILES, NB, BB, 16), jnp.float32),
        ),
        mesh=mesh,
        compiler_params=pltpu.CompilerParams(use_tc_tiling_on_sc=False),
        scratch_types=[
            pltpu.VMEM((NB, BB), jnp.int32),
            pltpu.VMEM((NB, BB), jnp.int32),
            pltpu.VMEM((BB, 16), jnp.float32),
            pltpu.VMEM((BB, 16), jnp.float32),
            pltpu.VMEM((BB, 16), jnp.float32),
            pltpu.VMEM((BB, 16), jnp.float32),
            pltpu.VMEM((BB, FW), jnp.float32),
            pltpu.VMEM((BB, FW), jnp.float32),
            pltpu.VMEM((BB, FW), jnp.float32),
            pltpu.VMEM_SHARED((NPAD, FW), jnp.float32),
            pltpu.SemaphoreType.DMA,
            pltpu.SemaphoreType.DMA,
            pltpu.SemaphoreType.DMA,
            pltpu.SemaphoreType.DMA,
            pltpu.SemaphoreType.DMA,
            pltpu.SemaphoreType.DMA,
        ],
    )
    def edge_kernel(feat, a_s, a_d, srcs, dsts, parts, wout,
                    idx_s, idx_d, ar, ad, wb0, wb1, rows0, rows1, zbuf, acc,
                    semr0, semr1, semw0, semw1, sems0, sems1):
        cid = lax.axis_index("c")
        sid = lax.axis_index("s")
        tid = cid * NS + sid
        fzero = jnp.zeros((16,), jnp.float32)

        def zb(j, _):
            for k2 in range(FW // 16):
                zbuf[j, k2 * 16:(k2 + 1) * 16] = fzero
            return 0
        lax.fori_loop(0, BB, zb, 0)

        pltpu.sync_copy(srcs.at[tid], idx_s)
        pltpu.sync_copy(dsts.at[tid], idx_d)

        def batch_a(b, _):
            ga = pltpu.async_copy(a_s.at[idx_s.at[b]], ar, semr0)
            gb = pltpu.async_copy(a_d.at[idx_d.at[b]], ad, semr1)
            ga.wait()
            gb.wait()

            def row_a(j, _):
                e = ar[j, :] + ad[j, :]
                e = jnp.where(e >= 0.0, e, e * 0.2)
                wb0[j, :] = jnp.exp(e)
                return 0
            lax.fori_loop(0, BB, row_a, 0, unroll=4)
            pltpu.sync_copy(wb0, wout.at[tid, b])
            return 0
        lax.fori_loop(0, NB, batch_a, 0)

        def head(hh, _):
            def zc(i, _):
                pltpu.sync_copy(zbuf, acc.at[pl.ds(sid * RPT + i * BB, BB)])
                return 0
            lax.fori_loop(0, NZ, zc, 0)
            plsc.subcore_barrier()

            hvec = jnp.full((16,), hh // 2, jnp.int32)

            def start(b, rbuf, wbuf, sr, sw):
                pltpu.async_copy(wout.at[tid, b], wbuf, sw)
                pltpu.async_copy(feat.at[hh].at[idx_s.at[b]], rbuf, sr)

            def scale(b, rbuf, wbuf, sr, sw):
                pltpu.make_async_copy(wout.at[tid, b], wbuf, sw).wait()
                pltpu.make_async_copy(
                    feat.at[hh].at[idx_s.at[b]], rbuf, sr).wait()

                def row_b(j, _):
                    wrow = wbuf[j, :]
                    ws = wrow.at[hvec].get(mode="promise_in_bounds")
                    for k2 in range(FW // 16):
                        sl = slice(k2 * 16, (k2 + 1) * 16)
                        rbuf[j, sl] = rbuf[j, sl] * ws
                    return 0
                lax.fori_loop(0, BB, row_b, 0, unroll=4)

            def scat_start(b, rbuf, ss):
                pltpu.async_copy(rbuf, acc.at[idx_d.at[b]], ss, add=True)

            def scat_wait(b, rbuf, ss):
                pltpu.make_async_copy(
                    rbuf, acc.at[idx_d.at[b]], ss).wait()

            start(0, rows0, wb0, semr0, semw0)

            def pair(bb, _):
                b0 = 2 * bb

                @pl.when(bb > 0)
                def _():
                    scat_wait(b0 - 1, rows1, sems1)
                start(b0 + 1, rows1, wb1, semr1, semw1)
                scale(b0, rows0, wb0, semr0, semw0)
                scat_start(b0, rows0, sems0)
                scale(b0 + 1, rows1, wb1, semr1, semw1)
                scat_wait(b0, rows0, sems0)

                @pl.when(b0 + 2 < NB)
                def _():
                    start(b0 + 2, rows0, wb0, semr0, semw0)
                scat_start(b0 + 1, rows1, sems1)
                return 0
            lax.fori_loop(0, NB // 2, pair, 0)
            scat_wait(NB - 1, rows1, sems1)
            plsc.subcore_barrier()

            def dc(i, _):
                base = sid * RPT + i * BB
                pltpu.sync_copy(acc.at[pl.ds(base, BB)], rows0)
                pltpu.sync_copy(rows0, parts.at[cid, hh].at[pl.ds(base, BB)])
                return 0
            lax.fori_loop(0, NZ, dc, 0)
            plsc.subcore_barrier()
            return 0
        lax.fori_loop(0, 2 * H, head, 0)

    return edge_kernel


def _feat1_k(x_ref, w_ref, o_ref):
    mm = jnp.dot(x_ref[...], w_ref[...], preferred_element_type=jnp.float32)
    col = lax.broadcasted_iota(jnp.int32, (BM, FW - FH), 1)
    pad1 = jnp.where(col == 0, 1.0, 0.0)
    pad0 = jnp.zeros((BM, FW - FH), jnp.float32)
    for hh in range(2 * HEADS):
        pad = pad1 if hh % 2 == 0 else pad0
        o_ref[hh] = jnp.concatenate(
            [mm[:, hh * FH:(hh + 1) * FH], pad], axis=1)


def _feat1(xp, w1c):
    return pl.pallas_call(
        _feat1_k,
        grid=(NPAD // BM,),
        in_specs=[
            pl.BlockSpec((BM, F), lambda i: (i, 0)),
            pl.BlockSpec((F, 2 * HEADS * FH), lambda i: (0, 0)),
        ],
        out_specs=pl.BlockSpec((2 * HEADS, BM, FW), lambda i: (0, i, 0)),
        out_shape=jax.ShapeDtypeStruct((2 * HEADS, NPAD, FW), jnp.float32),
    )(xp, w1c)


def _alpha_k(x_ref, ws_ref, wd_ref, os_ref, od_ref):
    os_ref[...] = jnp.dot(x_ref[...], ws_ref[...],
                          preferred_element_type=jnp.float32)
    od_ref[...] = jnp.dot(x_ref[...], wd_ref[...],
                          preferred_element_type=jnp.float32)


def _alphas(xp, wsp, wdp):
    return pl.pallas_call(
        _alpha_k,
        grid=(NPAD // BM,),
        in_specs=[
            pl.BlockSpec((BM, F), lambda i: (i, 0)),
            pl.BlockSpec((F, 16), lambda i: (0, 0)),
            pl.BlockSpec((F, 16), lambda i: (0, 0)),
        ],
        out_specs=[
            pl.BlockSpec((BM, 16), lambda i: (i, 0)),
            pl.BlockSpec((BM, 16), lambda i: (i, 0)),
        ],
        out_shape=[
            jax.ShapeDtypeStruct((NPAD, 16), jnp.float32),
            jax.ShapeDtypeStruct((NPAD, 16), jnp.float32),
        ],
    )(xp, wsp, wdp)


def _comb1_k(p_ref, b1_ref, w2_ref, a2s_ref, a2d_ref,
             of_ref, os_ref, od_ref):
    acc = jnp.zeros((BM, F), jnp.float32)
    for h in range(HEADS):
        e0 = p_ref[0, 2 * h] + p_ref[1, 2 * h]
        e1 = p_ref[0, 2 * h + 1] + p_ref[1, 2 * h + 1]
        den = e0[:, FH:FH + 1] + 1e-16
        v = jnp.concatenate([e0[:, :FH], e1[:, :FH]], axis=1) / den
        v = v + b1_ref[0, h]
        v = jnp.where(v > 0, v, jnp.exp(v) - 1.0)
        acc = acc + jnp.dot(v, w2_ref[h], preferred_element_type=jnp.float32)
    ones = jnp.ones((BM, 1), jnp.float32)
    z15 = jnp.zeros((BM, FW - FH - 1), jnp.float32)
    z16 = jnp.zeros((BM, FW - FH), jnp.float32)
    f_even = jnp.concatenate([acc[:, :FH], ones, z15], axis=1)
    f_odd = jnp.concatenate([acc[:, FH:], z16], axis=1)
    of_ref[...] = jnp.stack([f_even, f_odd], axis=0)
    s = jnp.sum(acc * a2s_ref[...], axis=1, keepdims=True)
    d = jnp.sum(acc * a2d_ref[...], axis=1, keepdims=True)
    za = jnp.zeros((BM, 15), jnp.float32)
    os_ref[...] = jnp.concatenate([s, za], axis=1)
    od_ref[...] = jnp.concatenate([d, za], axis=1)


def _comb1(parts1, b1r, w2r, a2s, a2d):
    return pl.pallas_call(
        _comb1_k,
        grid=(NPAD // BM,),
        in_specs=[
            pl.BlockSpec((NC, 2 * HEADS, BM, FW), lambda i: (0, 0, i, 0)),
            pl.BlockSpec((1, HEADS, F), lambda i: (0, 0, 0)),
            pl.BlockSpec((HEADS, F, F), lambda i: (0, 0, 0)),
            pl.BlockSpec((1, F), lambda i: (0, 0)),
            pl.BlockSpec((1, F), lambda i: (0, 0)),
        ],
        out_specs=[
            pl.BlockSpec((2, BM, FW), lambda i: (0, i, 0)),
            pl.BlockSpec((BM, 16), lambda i: (i, 0)),
            pl.BlockSpec((BM, 16), lambda i: (i, 0)),
        ],
        out_shape=[
            jax.ShapeDtypeStruct((2, NPAD, FW), jnp.float32),
            jax.ShapeDtypeStruct((NPAD, 16), jnp.float32),
            jax.ShapeDtypeStruct((NPAD, 16), jnp.float32),
        ],
    )(parts1, b1r, w2r, a2s, a2d)


def _pool_k(p_ref, b2_ref, o_ref):
    i = pl.program_id(0)
    e0 = p_ref[0, 0] + p_ref[1, 0]
    e1 = p_ref[0, 1] + p_ref[1, 1]
    den = e0[:, FH:FH + 1] + 1e-16
    v = jnp.concatenate([e0[:, :FH], e1[:, :FH]], axis=1) / den
    v = jnp.maximum(v + b2_ref[...], 0.0)
    rid = i * BM + lax.broadcasted_iota(jnp.int32, (BM, 1), 0)
    v = jnp.where(rid < N, v, 0.0)
    m = jnp.max(v.reshape(BM // 8, 8, F), axis=0)

    @pl.when(i == 0)
    def _():
        o_ref[...] = m

    @pl.when(i > 0)
    def _():
        o_ref[...] = jnp.maximum(o_ref[...], m)


def _pool(parts2, b2r):
    return pl.pallas_call(
        _pool_k,
        grid=(NPAD // BM,),
        in_specs=[
            pl.BlockSpec((NC, 2, BM, FW), lambda i: (0, 0, i, 0)),
            pl.BlockSpec((1, F), lambda i: (0, 0)),
        ],
        out_specs=pl.BlockSpec((8, F), lambda i: (0, 0)),
        out_shape=jax.ShapeDtypeStruct((8, F), jnp.float32),
    )(parts2, b2r)


def _mlp_k(g_ref, w0_ref, b0_ref, w1_ref, b1_ref, w2_ref, b2_ref,
           w3_ref, b3_ref, o_ref):
    g = jnp.max(g_ref[...], axis=0, keepdims=True)
    g = jnp.maximum(jnp.dot(g, w0_ref[...],
                            preferred_element_type=jnp.float32)
                    + b0_ref[...], 0.0)
    g = jnp.maximum(jnp.dot(g, w1_ref[...],
                            preferred_element_type=jnp.float32)
                    + b1_ref[...], 0.0)
    g = jnp.maximum(jnp.dot(g, w2_ref[...],
                            preferred_element_type=jnp.float32)
                    + b2_ref[...], 0.0)
    o_ref[...] = (jnp.dot(g, w3_ref[...], preferred_element_type=jnp.float32)
                  + b3_ref[...])


def _mlp(g8, fcg_w, fcg_b, fc1_w, fc1_b, fc2_w, fc2_b, out_w, out_b):
    return pl.pallas_call(
        _mlp_k,
        out_shape=jax.ShapeDtypeStruct((1, 128), jnp.float32),
    )(g8, fcg_w, fcg_b.reshape(1, -1), fc1_w, fc1_b.reshape(1, -1),
      fc2_w, fc2_b.reshape(1, -1), out_w, out_b.reshape(1, -1))


def kernel(x, edge_index, W1, a1_src, a1_dst, b1, W2, a2_src, a2_dst, b2,
           fcg_w, fcg_b, fc1_w, fc1_b, fc2_w, fc2_b, out_w, out_b):
    loop = jnp.arange(N, dtype=edge_index.dtype)
    ndum = EPAD - E - N
    dum = N + (jnp.arange(ndum, dtype=jnp.int32) % (NPAD - N))
    src = jnp.concatenate([edge_index[0], loop, dum])
    dst = jnp.concatenate([edge_index[1], loop, dum])
    srcs = src.reshape(NTILES, NB, BB)
    dsts = dst.reshape(NTILES, NB, BB)

    xp = jnp.pad(x, ((0, NPAD - N), (0, 0)))

    w1r = W1.reshape(F, HEADS, F).transpose(1, 0, 2)
    w1h = w1r.reshape(HEADS, F, 2, FH).transpose(0, 2, 1, 3)
    w1c = w1h.reshape(2 * HEADS, F, FH).transpose(1, 0, 2)
    w1c = w1c.reshape(F, 2 * HEADS * FH)
    ws1 = jnp.einsum("fhc,hc->fh", W1.reshape(F, HEADS, F), a1_src)
    wd1 = jnp.einsum("fhc,hc->fh", W1.reshape(F, HEADS, F), a1_dst)
    wsp = jnp.pad(ws1, ((0, 0), (0, 6)))
    wdp = jnp.pad(wd1, ((0, 0), (0, 6)))
    w2r = W2.reshape(HEADS, F, F)
    b1r = b1.reshape(1, HEADS, F)
    b2r = b2.reshape(1, F)

    feat1 = _feat1(xp, w1c)
    a_s1, a_d1 = _alphas(xp, wsp, wdp)
    parts1, _ = _make_edge_sc(HEADS)(feat1, a_s1, a_d1, srcs, dsts)

    feat2, a_s2, a_d2 = _comb1(parts1, b1r, w2r, a2_src, a2_dst)
    parts2, _ = _make_edge_sc(1)(feat2, a_s2, a_d2, srcs, dsts)

    g8 = _pool(parts2, b2r)
    return _mlp(g8, fcg_w, fcg_b, fc1_w, fc1_b, fc2_w, fc2_b, out_w, out_b)
